# Initial kernel scaffold; baseline (speedup 1.0000x reference)
import jax
import jax.numpy as jnp
from jax.experimental import pallas as pl


def kernel(node_ids, messages, timestamps):
    raise NotImplementedError("write your pallas kernel here")


# speedup vs baseline: 11.8877x; 11.8877x over previous
import jax
import jax.numpy as jnp
from jax import lax
from jax.experimental import pallas as pl
from jax.experimental.pallas import tpu as pltpu
from jax.experimental.pallas import tpu_sc as plsc

_N = 100000
_B = 16384
_D = 128
_NC = 2
_NS = 16
_L = 16
_NW = _NC * _NS
_SHARD = 3136
_NPAD = _NW * _SHARD
_NVB = _B // _L
_NVS = _SHARD // _L
_G = 128
_NBMAX = -(-_SHARD // _G)


def _sc_body(ids_hbm, msgs_hbm, ts_hbm, msg_out, has_out, ts_out,
             ids_v, ts_v, pos_t, has_t, tsl_t, win_pos, win_node, row_buf,
             sem):
    wid = lax.axis_index("s") * _NC + lax.axis_index("c")
    lo = wid * _SHARD
    iota = lax.iota(jnp.int32, _L)

    pltpu.sync_copy(ids_hbm, ids_v)
    pltpu.sync_copy(ts_hbm, ts_v)

    neg1 = jnp.full((_L,), -1, jnp.int32)

    def init_body(i, carry):
        pos_t[pl.ds(i * _L, _L)] = neg1
        return carry

    lax.fori_loop(0, _NVS, init_body, 0)

    def p1(i, carry):
        ids = ids_v[pl.ds(i * _L, _L)]
        inr = (ids >= lo) & (ids < lo + _SHARD)
        _, lastm = plsc.scan_count(ids, mask=inr)
        m = lastm & inr
        pos = jnp.full((_L,), i * _L, jnp.int32) + iota
        plsc.store_scatter(pos_t, [ids - lo], pos, mask=m)
        return carry

    lax.fori_loop(0, _NVB, p1, 0)

    def p2(i, off):
        posv = pos_t[pl.ds(i * _L, _L)]
        m = posv >= 0
        mi = jnp.where(m, 1, 0).astype(jnp.int32)
        has_t[pl.ds(i * _L, _L)] = mi
        tsv = plsc.load_gather(ts_v, [posv & (_B - 1)], mask=m)
        tsl_t[pl.ds(i * _L, _L)] = jnp.where(m, tsv, jnp.float32(0.0))
        c = plsc.cumsum(mi)
        widx = off + c - 1
        wrow = lax.shift_right_logical(widx, 7)
        wcol = widx & (_G - 1)
        plsc.store_scatter(win_pos, [wrow, wcol], posv, mask=m)
        node = jnp.full((_L,), lo + i * _L, jnp.int32) + iota
        plsc.store_scatter(win_node, [wrow, wcol], node, mask=m)
        return off + jnp.max(c)

    nwin = lax.fori_loop(0, _NVS, p2, jnp.int32(0))

    pltpu.sync_copy(has_t, has_out.at[pl.ds(lo, _SHARD)])
    pltpu.sync_copy(tsl_t, ts_out.at[pl.ds(lo, _SHARD)])

    nblk = (nwin + _G - 1) // _G
    end = nblk * _G
    for j in range(_G // _L):
        idx = nwin + j * _L + iota
        mpad = idx < end
        prow = lax.shift_right_logical(idx, 7)
        pcol = idx & (_G - 1)
        plsc.store_scatter(win_pos, [prow, pcol], neg1, mask=mpad)
        plsc.store_scatter(win_node, [prow, pcol], neg1, mask=mpad)

    def p3(b, carry):
        pltpu.async_copy(
            msgs_hbm.at[plsc.Indices(win_pos.at[b], ignored_value=-1)],
            row_buf, sem).wait()
        pltpu.async_copy(
            row_buf,
            msg_out.at[plsc.Indices(win_node.at[b], ignored_value=-1)],
            sem).wait()
        return carry

    lax.fori_loop(0, nblk, p3, 0)


_sc_call = pl.kernel(
    _sc_body,
    out_type=(
        jax.ShapeDtypeStruct((_NPAD,), jnp.int32),
        jax.ShapeDtypeStruct((_NPAD,), jnp.float32),
    ),
    mesh=plsc.VectorSubcoreMesh(core_axis_name="c", subcore_axis_name="s"),
    compiler_params=pltpu.CompilerParams(needs_layout_passes=False),
    scratch_types=[
        pltpu.VMEM((_B,), jnp.int32),
        pltpu.VMEM((_B,), jnp.float32),
        pltpu.VMEM((_SHARD,), jnp.int32),
        pltpu.VMEM((_SHARD,), jnp.int32),
        pltpu.VMEM((_SHARD,), jnp.float32),
        pltpu.VMEM((_NBMAX, _G), jnp.int32),
        pltpu.VMEM((_NBMAX, _G), jnp.int32),
        pltpu.VMEM((_G, _D), jnp.float32),
        pltpu.SemaphoreType.DMA,
    ],
)


@jax.jit
def kernel(node_ids, messages, timestamps):
    msg_ref = jax.new_ref(jnp.zeros((_N, _D), jnp.float32))
    has_pad, ts_pad = _sc_call(node_ids, messages, timestamps, msg_ref)
    return has_pad[:_N] != 0, jax.freeze(msg_ref), ts_pad[:_N]
